# baseline (speedup 1.0000x reference)
import functools

import jax
import jax.numpy as jnp
from jax import lax
from jax.experimental import pallas as pl
from jax.experimental.pallas import tpu as pltpu
from jax.experimental.pallas import tpu_sc as plsc

BATCH = 16384
MF_DIM = 8
MLP_HALF = 32
NUM_CORES = 2
NUM_SUBCORES = 16
NW = NUM_CORES * NUM_SUBCORES
BPW = BATCH // NW
CHUNK = 128
NCH = BPW // CHUNK

_f32 = jnp.float32


def _gather_table(tbl, idx_v, out, rb, sems, sem_w, base):
    def issue(c):
        slot = c % 2

        def body(j, _):
            u = idx_v[pl.ds(c * CHUNK + j, 16)][0]
            pltpu.make_async_copy(
                tbl.at[pl.ds(u, 1)],
                rb.at[pl.ds(slot * CHUNK + j, 1)],
                sems[slot],
            ).start()
            return _
        lax.fori_loop(0, CHUNK, body, None)

    def drain(c):
        def body(j, _):
            pltpu.make_async_copy(
                tbl.at[pl.ds(0, 1)], rb.at[pl.ds(0, 1)], sems[c % 2]
            ).wait()
            return _
        lax.fori_loop(0, CHUNK, body, None)

    def w_desc(c):
        slot = c % 2
        return pltpu.make_async_copy(
            rb.at[pl.ds(slot * CHUNK, CHUNK)],
            out.at[pl.ds(base + c * CHUNK, CHUNK)],
            sem_w[slot],
        )

    issue(0)
    issue(1)
    for c in range(NCH):
        drain(c)
        w_desc(c).start()
        if c + 2 < NCH:
            w_desc(c).wait()
            issue(c + 2)
    for c in range(NCH - 2, NCH):
        w_desc(c).wait()


def _sc_body(uidx_hbm, iidx_hbm, tu2, ti2,
             out_u, out_i,
             uidx_v, iidx_v, rb,
             s0, s1, w0, w1):
    wid = lax.axis_index("s") * NUM_CORES + lax.axis_index("c")
    base = wid * BPW
    pltpu.sync_copy(uidx_hbm.at[pl.ds(base, BPW)], uidx_v.at[pl.ds(0, BPW)])
    pltpu.sync_copy(iidx_hbm.at[pl.ds(base, BPW)], iidx_v.at[pl.ds(0, BPW)])
    sems = (s0, s1)
    sem_w = (w0, w1)
    _gather_table(tu2, uidx_v, out_u, rb, sems, sem_w, base)
    _gather_table(ti2, iidx_v, out_i, rb, sems, sem_w, base)


@functools.cache
def _sc_gather(d):
    return functools.partial(
        pl.kernel,
        out_type=(
            jax.ShapeDtypeStruct((BATCH, d), _f32),
            jax.ShapeDtypeStruct((BATCH, d), _f32),
        ),
        mesh=plsc.VectorSubcoreMesh(core_axis_name="c", subcore_axis_name="s"),
        scratch_types=[
            pltpu.VMEM((BPW + 16,), jnp.int32),
            pltpu.VMEM((BPW + 16,), jnp.int32),
            pltpu.VMEM((2 * CHUNK, d), _f32),
        ] + [pltpu.SemaphoreType.DMA] * 4,
        compiler_params=pltpu.CompilerParams(use_tc_tiling_on_sc=True),
    )(_sc_body)


BM = 2048


def _mlp_body(mu, mi, gu, gi, w1a, w1b, b1, w2, b2, w3, b3, wpg, wph, bp, out):
    h = jnp.maximum(mu[...] @ w1a[...] + mi[...] @ w1b[...] + b1[...], 0.0)
    h = jnp.maximum(h @ w2[...] + b2[...], 0.0)
    h = jnp.maximum(h @ w3[...] + b3[...], 0.0)
    g = gu[...] * gi[...]
    logit = g @ wpg[...] + h @ wph[...] + bp[...]
    out[...] = jax.nn.sigmoid(logit[:, 0])


def kernel(user_indices, item_indices, ue_gmf, ie_gmf, ue_mlp, ie_mlp,
           W1, b1, W2, b2, W3, b3, Wp, bp):
    uidx = user_indices.astype(jnp.int32)
    iidx = item_indices.astype(jnp.int32)
    mu, mi = _sc_gather(MLP_HALF)(uidx, iidx, ue_mlp, ie_mlp)
    gu, gi = _sc_gather(MF_DIM)(uidx, iidx, ue_gmf, ie_gmf)

    grid = BATCH // BM
    row_spec = lambda d: pl.BlockSpec((BM, d), lambda i: (i, 0))
    full = lambda a: pl.BlockSpec(a.shape, lambda i: (0,) * a.ndim)
    w1a, w1b = W1[:MLP_HALF], W1[MLP_HALF:]
    wpg, wph = Wp[:MF_DIM], Wp[MF_DIM:]
    b1r, b2r, b3r, bpr = (b.reshape(1, -1) for b in (b1, b2, b3, bp))
    out = pl.pallas_call(
        _mlp_body,
        grid=(grid,),
        in_specs=[row_spec(MLP_HALF), row_spec(MLP_HALF),
                  row_spec(MF_DIM), row_spec(MF_DIM),
                  full(w1a), full(w1b), full(b1r), full(W2), full(b2r),
                  full(W3), full(b3r), full(wpg), full(wph), full(bpr)],
        out_specs=pl.BlockSpec((BM,), lambda i: (i,)),
        out_shape=jax.ShapeDtypeStruct((BATCH,), _f32),
    )(mu, mi, gu, gi, w1a, w1b, b1r, W2, b2r, W3, b3r, wpg, wph, bpr)
    return out
